# Initial kernel scaffold; baseline (speedup 1.0000x reference)
import jax
import jax.numpy as jnp
from jax.experimental import pallas as pl


def kernel(x, edge_index, batch, W1, b1, W2, b2, gamma, beta, Wa, Va, ba, Wf1, bf1, Wf2, bf2, Wf3, bf3, Wf4, bf4):
    raise NotImplementedError("write your pallas kernel here")


# speedup vs baseline: 8.5730x; 8.5730x over previous
import functools

import jax
import jax.numpy as jnp
from jax import lax
from jax.experimental import pallas as pl
from jax.experimental.pallas import tpu as pltpu
from jax.experimental.pallas import tpu_sc as plsc

N = 10000
E = 320000
F_IN = 128
DIM = 100
G = 64

NC = 2
NS = 16
K = 128
CH = 79
E_PAD = NC * NS * CH * K
N_PAD = 10112
RPT = N_PAD // NS
D = 128


def _prop_body(table, rowi, coli, zrs, out, acc, ridx, cidx, rows, sem):
    c = lax.axis_index("c")
    s = lax.axis_index("s")
    pltpu.sync_copy(zrs.at[pl.ds(s * RPT, RPT)], acc.at[pl.ds(s * RPT, RPT)])
    plsc.subcore_barrier()
    base = (c * NS + s) * CH * K

    def chunk(i, carry):
        off = base + i * K
        pltpu.sync_copy(rowi.at[pl.ds(off, K)], ridx)
        pltpu.sync_copy(coli.at[pl.ds(off, K)], cidx)
        pltpu.async_copy(table.at[ridx], rows, sem).wait()
        pltpu.sync_copy(rows, acc.at[cidx], add=True)
        return carry

    lax.fori_loop(0, CH, chunk, 0)
    plsc.subcore_barrier()
    pltpu.sync_copy(acc.at[pl.ds(s * RPT, RPT)], out.at[c, pl.ds(s * RPT, RPT)])


@functools.cache
def _get_prop(width):
    mesh = plsc.VectorSubcoreMesh(core_axis_name="c", subcore_axis_name="s",
                                  num_cores=NC, num_subcores=NS)
    return pl.kernel(
        _prop_body,
        out_type=jax.ShapeDtypeStruct((NC, N_PAD, width), jnp.float32),
        mesh=mesh,
        scratch_types=[
            pltpu.VMEM_SHARED((N_PAD, width), jnp.float32),
            pltpu.VMEM((K,), jnp.int32),
            pltpu.VMEM((K,), jnp.int32),
            pltpu.VMEM((K, width), jnp.float32),
            pltpu.SemaphoreType.DMA,
        ],
    )


def _prop(table, rowi, coli, zrs):
    return _get_prop(table.shape[1])(table, rowi, coli, zrs)


def _deg(coli, ones_table, zrs):
    return _prop(ones_table, coli, coli, zrs)


def _rowmask():
    return (lax.broadcasted_iota(jnp.int32, (N_PAD, 1), 0) < N).astype(jnp.float32)


def _tca_body(deg2, x, xs_o, dinv1_o, dinv2_o):
    deg = deg2[0, :, 0:1] + deg2[1, :, 0:1]
    dinv1 = 1.0 / jnp.sqrt(deg + 1.0)
    dinv2 = jnp.where(deg > 0, 1.0 / jnp.sqrt(jnp.maximum(deg, 1.0)), 0.0)
    xs_o[:] = dinv1 * x[:]
    dinv1_o[:] = dinv1
    dinv2_o[:] = dinv2


_tca = pl.pallas_call(
    _tca_body,
    out_shape=(
        jax.ShapeDtypeStruct((N_PAD, D), jnp.float32),
        jax.ShapeDtypeStruct((N_PAD, 1), jnp.float32),
        jax.ShapeDtypeStruct((N_PAD, 1), jnp.float32),
    ),
)


def _tcb_body(acc1, xs, dinv1, w1, b1, hs_o):
    hp = dinv1[:] * (acc1[0] + acc1[1] + xs[:])
    h = jnp.maximum(jnp.dot(hp, w1[:], preferred_element_type=jnp.float32)
                    + b1[:], 0.0)
    hs_o[:] = dinv1[:] * h * _rowmask()


_tcb = pl.pallas_call(
    _tcb_body,
    out_shape=jax.ShapeDtypeStruct((N_PAD, D), jnp.float32),
)


def _tcc_body(acc2, hs, dinv1, dinv2, w2, b2, gam, bet, wa, ts_o, hbn_o):
    h2p = dinv1[:] * (acc2[0] + acc2[1] + hs[:])
    h2 = jnp.maximum(jnp.dot(h2p, w2[:], preferred_element_type=jnp.float32)
                     + b2[:], 0.0)
    h2 = h2 * _rowmask()
    mean = jnp.sum(h2, axis=0, keepdims=True) * (1.0 / N)
    dev = (h2 - mean) * _rowmask()
    var = jnp.sum(dev * dev, axis=0, keepdims=True) * (1.0 / N)
    hbn = ((h2 - mean) * (1.0 / jnp.sqrt(var + 1e-5)) * gam[:] + bet[:]) * _rowmask()
    hbn_o[:] = hbn
    t0 = jnp.dot(hbn, wa[:], preferred_element_type=jnp.float32)
    ts_o[:] = dinv2[:] * t0


_tcc = pl.pallas_call(
    _tcc_body,
    out_shape=(
        jax.ShapeDtypeStruct((N_PAD, D), jnp.float32),
        jax.ShapeDtypeStruct((N_PAD, D), jnp.float32),
    ),
)


def _tcd_body(acc3, hbn, dinv2, va, ba, bat, wf1, bf1, wf2, bf2, wf3, bf3,
              wf4, bf4, out_o):
    t = dinv2[:] * (acc3[0] + acc3[1])
    hf = t + jnp.dot(hbn[:], va[:], preferred_element_type=jnp.float32) + ba[:]
    hf = jnp.maximum(hf, 0.0) * _rowmask()
    oneh = (bat[:] == lax.broadcasted_iota(jnp.int32, (N_PAD, G), 1))
    oneh = oneh.astype(jnp.float32)
    g = lax.dot_general(oneh, hf, (((0,), (0,)), ((), ())),
                        preferred_element_type=jnp.float32,
                        precision=lax.Precision.HIGHEST)
    g = jnp.maximum(jnp.dot(g, wf1[:], preferred_element_type=jnp.float32)
                    + bf1[:], 0.0)
    g = jnp.maximum(jnp.dot(g, wf2[:], preferred_element_type=jnp.float32)
                    + bf2[:], 0.0)
    g = jnp.maximum(jnp.dot(g, wf3[:], preferred_element_type=jnp.float32)
                    + bf3[:], 0.0)
    out_o[:] = jnp.dot(g, wf4[:], preferred_element_type=jnp.float32) + bf4[:]


_tcd = pl.pallas_call(
    _tcd_body,
    out_shape=jax.ShapeDtypeStruct((G, 1), jnp.float32),
)


def _pad2(a, r, c):
    return jnp.pad(a, ((0, r - a.shape[0]), (0, c - a.shape[1])))


def kernel(x, edge_index, batch, W1, b1, W2, b2, gamma, beta, Wa, Va, ba,
           Wf1, bf1, Wf2, bf2, Wf3, bf3, Wf4, bf4):
    row = edge_index[0].astype(jnp.int32)
    col = edge_index[1].astype(jnp.int32)
    pad = jnp.full((E_PAD - E,), N, jnp.int32)
    rowp = jnp.concatenate([row, pad])
    colp = jnp.concatenate([col, pad])

    x_pad = _pad2(x, N_PAD, F_IN)
    w1p = _pad2(W1, F_IN, D)
    w2p = _pad2(W2, D, D)
    wap = _pad2(Wa, D, D)
    vap = _pad2(Va, D, D)
    b1p = _pad2(b1[None, :], 1, D)
    b2p = _pad2(b2[None, :], 1, D)
    gamp = _pad2(gamma[None, :], 1, D)
    betp = _pad2(beta[None, :], 1, D)
    bap = _pad2(ba[None, :], 1, D)
    wf1p = _pad2(Wf1, D, 256)
    bf1p = _pad2(bf1[None, :], 1, 256)
    wf2p = _pad2(Wf2, 256, 384)
    bf2p = _pad2(bf2[None, :], 1, 384)
    wf3p = _pad2(Wf3, 384, 256)
    bf3p = _pad2(bf3[None, :], 1, 256)
    wf4p = _pad2(Wf4, 256, 1)
    bf4p = bf4[None, :]
    batp = jnp.pad(batch.astype(jnp.int32), (0, N_PAD - N),
                   constant_values=G)[:, None]

    zerosD = jnp.zeros((N_PAD, D), jnp.float32)
    onesD = jnp.ones((N_PAD, D), jnp.float32)

    deg2 = _deg(colp, onesD, zerosD)
    xs, dinv1, dinv2 = _tca(deg2, x_pad)
    acc1 = _prop(xs, rowp, colp, zerosD)
    hs = _tcb(acc1, xs, dinv1, w1p, b1p)
    acc2 = _prop(hs, rowp, colp, zerosD)
    ts, hbn = _tcc(acc2, hs, dinv1, dinv2, w2p, b2p, gamp, betp, wap)
    acc3 = _prop(ts, rowp, colp, zerosD)
    return _tcd(acc3, hbn, dinv2, vap, bap, batp, wf1p, bf1p, wf2p, bf2p,
                wf3p, bf3p, wf4p, bf4p)
